# baseline (speedup 1.0000x reference)
import functools
import jax
import jax.numpy as jnp
from jax import lax
from jax.experimental import pallas as pl
from jax.experimental.pallas import tpu as pltpu
from jax.experimental.pallas import tpu_sc as plsc

NC = 2
NS = 16
NW = NC * NS
K = 128
ZR = 32


def _fill_const(buf, rows, val):
    @pl.loop(0, rows)
    def _(r):
        @pl.loop(0, 8)
        def _(cc):
            buf[pl.ds(r, 1), pl.ds(cc * 16, 16)] = jnp.full((1, 16), val, jnp.float32)


def _sc_deg_body(ch, slab_rows, dstm_hbm, out_hbm, idx_d, ones_v, zbuf, acc):
    cid = lax.axis_index("c")
    sid = lax.axis_index("s")
    wid = cid * NS + sid
    slab = sid * slab_rows
    n_acc = slab_rows * NS

    _fill_const(zbuf, ZR, 0.0)
    _fill_const(ones_v, K, 1.0)

    @pl.loop(0, slab_rows // ZR)
    def _(i):
        pltpu.sync_copy(zbuf, acc.at[pl.ds(slab + i * ZR, ZR)])

    pltpu.sync_copy(dstm_hbm.at[pl.ds(wid * ch, ch)], idx_d)
    plsc.subcore_barrier()

    @pl.loop(0, ch)
    def _(c):
        pltpu.sync_copy(ones_v, acc.at[idx_d.at[c]], add=True)

    plsc.subcore_barrier()
    pltpu.sync_copy(acc.at[pl.ds(slab, slab_rows)],
                    out_hbm.at[pl.ds(cid * n_acc + slab, slab_rows)])


def _sc_pass_body(ch0, ch1, slab_rows, z_hbm, srcm_hbm, dstm_hbm, out_hbm,
                  is0, is1, id0, id1, rows0, rows1, zbuf, acc, sem0, sem1):
    cid = lax.axis_index("c")
    sid = lax.axis_index("s")
    slab = sid * slab_rows
    n_acc = slab_rows * NS
    ch = lax.select(cid == 0, ch0, ch1)
    base = lax.select(cid == 0, sid * ch0, NS * ch0 + sid * ch1)

    _fill_const(zbuf, ZR, 0.0)

    @pl.loop(0, slab_rows // ZR)
    def _(i):
        pltpu.sync_copy(zbuf, acc.at[pl.ds(slab + i * ZR, ZR)])

    pltpu.sync_copy(srcm_hbm.at[base], is0)
    pltpu.sync_copy(dstm_hbm.at[base], id0)
    pltpu.sync_copy(srcm_hbm.at[base + 1], is1)
    pltpu.sync_copy(dstm_hbm.at[base + 1], id1)
    pltpu.async_copy(z_hbm.at[is0], rows0, sem0)
    pltpu.async_copy(z_hbm.at[is1], rows1, sem1)

    plsc.subcore_barrier()

    @pl.loop(0, ch, step=2)
    def _(c):
        pltpu.make_async_copy(z_hbm.at[is0], rows0, sem0).wait()
        pltpu.sync_copy(rows0, acc.at[id0], add=True)

        @pl.when(c + 2 < ch)
        def _():
            pltpu.sync_copy(srcm_hbm.at[base + c + 2], is0)
            pltpu.sync_copy(dstm_hbm.at[base + c + 2], id0)
            pltpu.async_copy(z_hbm.at[is0], rows0, sem0)

        pltpu.make_async_copy(z_hbm.at[is1], rows1, sem1).wait()
        pltpu.sync_copy(rows1, acc.at[id1], add=True)

        @pl.when(c + 3 < ch)
        def _():
            pltpu.sync_copy(srcm_hbm.at[base + c + 3], is1)
            pltpu.sync_copy(dstm_hbm.at[base + c + 3], id1)
            pltpu.async_copy(z_hbm.at[is1], rows1, sem1)

    plsc.subcore_barrier()
    pltpu.sync_copy(acc.at[pl.ds(slab, slab_rows)],
                    out_hbm.at[pl.ds(cid * n_acc + slab, slab_rows)])


SLOW_CORE_FRAC = 0.23


@functools.cache
def _make_sc_kernels(n, e, d):
    chunks_total = -(-(-(-e // K)) // (NW * 8)) * (NW * 8)
    ch = chunks_total // NW
    ch_sum = chunks_total // NS
    ch0 = max(2, int(ch_sum * SLOW_CORE_FRAC) // 2 * 2)
    ch1 = ch_sum - ch0
    e_pad = chunks_total * K
    n_acc = -(-(n + 1) // (NS * ZR)) * (NS * ZR)
    slab_rows = n_acc // NS
    mesh = plsc.VectorSubcoreMesh(core_axis_name="c", subcore_axis_name="s")

    deg_kernel = pl.kernel(
        functools.partial(_sc_deg_body, ch, slab_rows),
        out_type=jax.ShapeDtypeStruct((NC * n_acc, 128), jnp.float32),
        mesh=mesh,
        scratch_types=[
            pltpu.VMEM((ch, K), jnp.int32),
            pltpu.VMEM((K, 128), jnp.float32),
            pltpu.VMEM((ZR, 128), jnp.float32),
            pltpu.VMEM_SHARED((n_acc, 128), jnp.float32),
        ],
    )

    pass_kernel = pl.kernel(
        functools.partial(_sc_pass_body, ch0, ch1, slab_rows),
        out_type=jax.ShapeDtypeStruct((NC * n_acc, d), jnp.float32),
        mesh=mesh,
        scratch_types=[
            pltpu.VMEM((K,), jnp.int32),
            pltpu.VMEM((K,), jnp.int32),
            pltpu.VMEM((K,), jnp.int32),
            pltpu.VMEM((K,), jnp.int32),
            pltpu.VMEM((K, d), jnp.float32),
            pltpu.VMEM((K, d), jnp.float32),
            pltpu.VMEM((ZR, 128), jnp.float32),
            pltpu.VMEM_SHARED((n_acc, d), jnp.float32),
            pltpu.SemaphoreType.DMA,
            pltpu.SemaphoreType.DMA,
        ],
    )
    return deg_kernel, pass_kernel, e_pad, n_acc


BLK = 2048


def _tc_mm1_body(x_ref, w_ref, o_ref):
    o_ref[...] = jnp.dot(x_ref[...], w_ref[...], preferred_element_type=jnp.float32)


def _tc_scale1_body(pa_ref, pb_ref, h1_ref, dinv_ref, z1_ref):
    dinv = lax.rsqrt(pa_ref[...] + pb_ref[...] + 1.0)
    dinv_ref[...] = dinv
    z1_ref[...] = dinv * h1_ref[...]


def _tc_mid_body(sa_ref, sb_ref, dinv_ref, h1_ref, b1_ref, hid_ref, z2_ref):
    dinv = dinv_ref[...]
    agg = dinv * (sa_ref[...] + sb_ref[...]) + dinv * dinv * h1_ref[...]
    hid = jnp.maximum(agg + b1_ref[...], 0.0)
    hid_ref[...] = hid
    z2_ref[...] = dinv * hid


def _tc_out_body(sa_ref, sb_ref, dinv_ref, hid_ref, wmu_ref, wlv_ref,
                 bmu_ref, blv_ref, mu_ref, lv_ref):
    dinv = dinv_ref[...]
    agg = dinv * (sa_ref[...] + sb_ref[...]) + dinv * dinv * hid_ref[...]
    mu_ref[...] = jnp.dot(agg, wmu_ref[...], preferred_element_type=jnp.float32) + bmu_ref[...]
    lv_ref[...] = jnp.dot(agg, wlv_ref[...], preferred_element_type=jnp.float32) + blv_ref[...]


def _row_spec(off_blocks):
    return pl.BlockSpec((BLK, 128), lambda i, _o=off_blocks: (i + _o, 0))


def _full_spec(shape):
    return pl.BlockSpec(shape, lambda i: (0, 0))


def kernel(x, adj, W1, b1, W_mu, b_mu, W_lv, b_lv):
    n, d = x.shape
    e = adj.shape[1]
    deg_kernel, pass_kernel, e_pad, n_acc = _make_sc_kernels(n, e, d)
    grid = (n_acc // BLK,)
    nb = n_acc // BLK
    h2 = W_mu.shape[1]

    src, dst = adj[0], adj[1]
    pad = e_pad - e
    srcm = jnp.concatenate([src, jnp.zeros((pad,), jnp.int32)]).reshape(e_pad // K, K)
    dstm = jnp.concatenate([dst, jnp.full((pad,), n, jnp.int32)]).reshape(e_pad // K, K)
    xp = jnp.concatenate([x, jnp.zeros((n_acc - n, d), jnp.float32)])

    degp = deg_kernel(dstm)
    h1 = pl.pallas_call(
        _tc_mm1_body, grid=grid,
        in_specs=[_row_spec(0), _full_spec((d, d))],
        out_specs=_row_spec(0),
        out_shape=jax.ShapeDtypeStruct((n_acc, d), jnp.float32),
    )(xp, W1)

    dinv, z1 = pl.pallas_call(
        _tc_scale1_body, grid=grid,
        in_specs=[_row_spec(0), _row_spec(nb), _row_spec(0)],
        out_specs=[_row_spec(0), _row_spec(0)],
        out_shape=[jax.ShapeDtypeStruct((n_acc, d), jnp.float32),
                   jax.ShapeDtypeStruct((n_acc, d), jnp.float32)],
    )(degp, degp, h1)

    s1 = pass_kernel(z1, srcm, dstm)

    hidden, z2 = pl.pallas_call(
        _tc_mid_body, grid=grid,
        in_specs=[_row_spec(0), _row_spec(nb), _row_spec(0), _row_spec(0),
                  pl.BlockSpec((1, d), lambda i: (0, 0))],
        out_specs=[_row_spec(0), _row_spec(0)],
        out_shape=[jax.ShapeDtypeStruct((n_acc, d), jnp.float32),
                   jax.ShapeDtypeStruct((n_acc, d), jnp.float32)],
    )(s1, s1, dinv, h1, b1.reshape(1, d))

    s2 = pass_kernel(z2, srcm, dstm)

    mu_f, lv_f = pl.pallas_call(
        _tc_out_body, grid=grid,
        in_specs=[_row_spec(0), _row_spec(nb), _row_spec(0), _row_spec(0),
                  _full_spec((d, h2)), _full_spec((d, h2)),
                  pl.BlockSpec((1, h2), lambda i: (0, 0)),
                  pl.BlockSpec((1, h2), lambda i: (0, 0))],
        out_specs=[pl.BlockSpec((BLK, h2), lambda i: (i, 0)),
                   pl.BlockSpec((BLK, h2), lambda i: (i, 0))],
        out_shape=[jax.ShapeDtypeStruct((n_acc, h2), jnp.float32),
                   jax.ShapeDtypeStruct((n_acc, h2), jnp.float32)],
    )(s2, s2, dinv, hidden, W_mu, W_lv, b_mu.reshape(1, h2), b_lv.reshape(1, h2))

    return (mu_f[:n], lv_f[:n])


# speedup vs baseline: 1.1827x; 1.1827x over previous
import functools
import jax
import jax.numpy as jnp
from jax import lax
from jax.experimental import pallas as pl
from jax.experimental.pallas import tpu as pltpu
from jax.experimental.pallas import tpu_sc as plsc

NC = 2
NS = 16
NW = NC * NS
K = 128
ZR = 32


def _fill_const(buf, rows, val):
    @pl.loop(0, rows)
    def _(r):
        @pl.loop(0, 8)
        def _(cc):
            buf[pl.ds(r, 1), pl.ds(cc * 16, 16)] = jnp.full((1, 16), val, jnp.float32)


def _sc_deg_body(ch, slab_rows, dstm_hbm, out_hbm, idx_d, ones_v, zbuf, acc):
    cid = lax.axis_index("c")
    sid = lax.axis_index("s")
    wid = cid * NS + sid
    slab = sid * slab_rows
    n_acc = slab_rows * NS

    _fill_const(zbuf, ZR, 0.0)
    _fill_const(ones_v, K, 1.0)

    @pl.loop(0, slab_rows // ZR)
    def _(i):
        pltpu.sync_copy(zbuf, acc.at[pl.ds(slab + i * ZR, ZR)])

    pltpu.sync_copy(dstm_hbm.at[pl.ds(wid * ch, ch)], idx_d)
    plsc.subcore_barrier()

    @pl.loop(0, ch)
    def _(c):
        pltpu.sync_copy(ones_v, acc.at[idx_d.at[c]], add=True)

    plsc.subcore_barrier()
    pltpu.sync_copy(acc.at[pl.ds(slab, slab_rows)],
                    out_hbm.at[pl.ds(cid * n_acc + slab, slab_rows)])


def _sc_pass_body(ch0, ch1, slab_rows, z_hbm, srcm_hbm, dstm_hbm, out_hbm,
                  is0, is1, id0, id1, rows0, rows1, zbuf, acc, sem0, sem1):
    cid = lax.axis_index("c")
    sid = lax.axis_index("s")
    slab = sid * slab_rows
    n_acc = slab_rows * NS
    ch = lax.select(cid == 0, ch0, ch1)
    base = lax.select(cid == 0, sid * ch0, NS * ch0 + sid * ch1)

    _fill_const(zbuf, ZR, 0.0)

    @pl.loop(0, slab_rows // ZR)
    def _(i):
        pltpu.sync_copy(zbuf, acc.at[pl.ds(slab + i * ZR, ZR)])

    pltpu.sync_copy(srcm_hbm.at[base], is0)
    pltpu.sync_copy(dstm_hbm.at[base], id0)
    pltpu.sync_copy(srcm_hbm.at[base + 1], is1)
    pltpu.sync_copy(dstm_hbm.at[base + 1], id1)
    pltpu.async_copy(z_hbm.at[is0], rows0, sem0)
    pltpu.async_copy(z_hbm.at[is1], rows1, sem1)

    plsc.subcore_barrier()

    @pl.loop(0, ch, step=2)
    def _(c):
        pltpu.make_async_copy(z_hbm.at[is0], rows0, sem0).wait()
        pltpu.sync_copy(rows0, acc.at[id0], add=True)

        @pl.when(c + 2 < ch)
        def _():
            pltpu.sync_copy(srcm_hbm.at[base + c + 2], is0)
            pltpu.sync_copy(dstm_hbm.at[base + c + 2], id0)
            pltpu.async_copy(z_hbm.at[is0], rows0, sem0)

        pltpu.make_async_copy(z_hbm.at[is1], rows1, sem1).wait()
        pltpu.sync_copy(rows1, acc.at[id1], add=True)

        @pl.when(c + 3 < ch)
        def _():
            pltpu.sync_copy(srcm_hbm.at[base + c + 3], is1)
            pltpu.sync_copy(dstm_hbm.at[base + c + 3], id1)
            pltpu.async_copy(z_hbm.at[is1], rows1, sem1)

    plsc.subcore_barrier()
    pltpu.sync_copy(acc.at[pl.ds(slab, slab_rows)],
                    out_hbm.at[pl.ds(cid * n_acc + slab, slab_rows)])


SLOW_CORE_FRAC = 0.23


@functools.cache
def _make_sc_kernels(n, e, d):
    chunks_total = -(-(-(-e // K)) // (NW * 8)) * (NW * 8)
    ch = chunks_total // NW
    ch_sum = chunks_total // NS
    ch1 = max(2, int(ch_sum * SLOW_CORE_FRAC) // 2 * 2)
    ch0 = ch_sum - ch1
    e_pad = chunks_total * K
    n_acc = -(-(n + 1) // (NS * ZR)) * (NS * ZR)
    slab_rows = n_acc // NS
    mesh = plsc.VectorSubcoreMesh(core_axis_name="c", subcore_axis_name="s")

    deg_kernel = pl.kernel(
        functools.partial(_sc_deg_body, ch, slab_rows),
        out_type=jax.ShapeDtypeStruct((NC * n_acc, 128), jnp.float32),
        mesh=mesh,
        scratch_types=[
            pltpu.VMEM((ch, K), jnp.int32),
            pltpu.VMEM((K, 128), jnp.float32),
            pltpu.VMEM((ZR, 128), jnp.float32),
            pltpu.VMEM_SHARED((n_acc, 128), jnp.float32),
        ],
    )

    pass_kernel = pl.kernel(
        functools.partial(_sc_pass_body, ch0, ch1, slab_rows),
        out_type=jax.ShapeDtypeStruct((NC * n_acc, d), jnp.float32),
        mesh=mesh,
        scratch_types=[
            pltpu.VMEM((K,), jnp.int32),
            pltpu.VMEM((K,), jnp.int32),
            pltpu.VMEM((K,), jnp.int32),
            pltpu.VMEM((K,), jnp.int32),
            pltpu.VMEM((K, d), jnp.float32),
            pltpu.VMEM((K, d), jnp.float32),
            pltpu.VMEM((ZR, 128), jnp.float32),
            pltpu.VMEM_SHARED((n_acc, d), jnp.float32),
            pltpu.SemaphoreType.DMA,
            pltpu.SemaphoreType.DMA,
        ],
    )
    return deg_kernel, pass_kernel, e_pad, n_acc


BLK = 2048


def _tc_mm1_body(x_ref, w_ref, o_ref):
    o_ref[...] = jnp.dot(x_ref[...], w_ref[...], preferred_element_type=jnp.float32)


def _tc_scale1_body(pa_ref, pb_ref, h1_ref, dinv_ref, z1_ref):
    dinv = lax.rsqrt(pa_ref[...] + pb_ref[...] + 1.0)
    dinv_ref[...] = dinv
    z1_ref[...] = dinv * h1_ref[...]


def _tc_mid_body(sa_ref, sb_ref, dinv_ref, h1_ref, b1_ref, hid_ref, z2_ref):
    dinv = dinv_ref[...]
    agg = dinv * (sa_ref[...] + sb_ref[...]) + dinv * dinv * h1_ref[...]
    hid = jnp.maximum(agg + b1_ref[...], 0.0)
    hid_ref[...] = hid
    z2_ref[...] = dinv * hid


def _tc_out_body(sa_ref, sb_ref, dinv_ref, hid_ref, wmu_ref, wlv_ref,
                 bmu_ref, blv_ref, mu_ref, lv_ref):
    dinv = dinv_ref[...]
    agg = dinv * (sa_ref[...] + sb_ref[...]) + dinv * dinv * hid_ref[...]
    mu_ref[...] = jnp.dot(agg, wmu_ref[...], preferred_element_type=jnp.float32) + bmu_ref[...]
    lv_ref[...] = jnp.dot(agg, wlv_ref[...], preferred_element_type=jnp.float32) + blv_ref[...]


def _row_spec(off_blocks):
    return pl.BlockSpec((BLK, 128), lambda i, _o=off_blocks: (i + _o, 0))


def _full_spec(shape):
    return pl.BlockSpec(shape, lambda i: (0, 0))


def kernel(x, adj, W1, b1, W_mu, b_mu, W_lv, b_lv):
    n, d = x.shape
    e = adj.shape[1]
    deg_kernel, pass_kernel, e_pad, n_acc = _make_sc_kernels(n, e, d)
    grid = (n_acc // BLK,)
    nb = n_acc // BLK
    h2 = W_mu.shape[1]

    src, dst = adj[0], adj[1]
    pad = e_pad - e
    srcm = jnp.concatenate([src, jnp.zeros((pad,), jnp.int32)]).reshape(e_pad // K, K)
    dstm = jnp.concatenate([dst, jnp.full((pad,), n, jnp.int32)]).reshape(e_pad // K, K)
    xp = jnp.concatenate([x, jnp.zeros((n_acc - n, d), jnp.float32)])

    degp = deg_kernel(dstm)
    h1 = pl.pallas_call(
        _tc_mm1_body, grid=grid,
        in_specs=[_row_spec(0), _full_spec((d, d))],
        out_specs=_row_spec(0),
        out_shape=jax.ShapeDtypeStruct((n_acc, d), jnp.float32),
    )(xp, W1)

    dinv, z1 = pl.pallas_call(
        _tc_scale1_body, grid=grid,
        in_specs=[_row_spec(0), _row_spec(nb), _row_spec(0)],
        out_specs=[_row_spec(0), _row_spec(0)],
        out_shape=[jax.ShapeDtypeStruct((n_acc, d), jnp.float32),
                   jax.ShapeDtypeStruct((n_acc, d), jnp.float32)],
    )(degp, degp, h1)

    s1 = pass_kernel(z1, srcm, dstm)

    hidden, z2 = pl.pallas_call(
        _tc_mid_body, grid=grid,
        in_specs=[_row_spec(0), _row_spec(nb), _row_spec(0), _row_spec(0),
                  pl.BlockSpec((1, d), lambda i: (0, 0))],
        out_specs=[_row_spec(0), _row_spec(0)],
        out_shape=[jax.ShapeDtypeStruct((n_acc, d), jnp.float32),
                   jax.ShapeDtypeStruct((n_acc, d), jnp.float32)],
    )(s1, s1, dinv, h1, b1.reshape(1, d))

    s2 = pass_kernel(z2, srcm, dstm)

    mu_f, lv_f = pl.pallas_call(
        _tc_out_body, grid=grid,
        in_specs=[_row_spec(0), _row_spec(nb), _row_spec(0), _row_spec(0),
                  _full_spec((d, h2)), _full_spec((d, h2)),
                  pl.BlockSpec((1, h2), lambda i: (0, 0)),
                  pl.BlockSpec((1, h2), lambda i: (0, 0))],
        out_specs=[pl.BlockSpec((BLK, h2), lambda i: (i, 0)),
                   pl.BlockSpec((BLK, h2), lambda i: (i, 0))],
        out_shape=[jax.ShapeDtypeStruct((n_acc, h2), jnp.float32),
                   jax.ShapeDtypeStruct((n_acc, h2), jnp.float32)],
    )(s2, s2, dinv, hidden, W_mu, W_lv, b_mu.reshape(1, h2), b_lv.reshape(1, h2))

    return (mu_f[:n], lv_f[:n])


# speedup vs baseline: 2.9494x; 2.4939x over previous
import functools
import jax
import jax.numpy as jnp
from jax import lax
from jax.experimental import pallas as pl
from jax.experimental.pallas import tpu as pltpu
from jax.experimental.pallas import tpu_sc as plsc

NC = 2
NS = 16
NW = NC * NS
K = 128
ZR = 32


def _fill_const(buf, rows, val):
    @pl.loop(0, rows)
    def _(r):
        @pl.loop(0, 8)
        def _(cc):
            buf[pl.ds(r, 1), pl.ds(cc * 16, 16)] = jnp.full((1, 16), val, jnp.float32)


def _sc_deg_body(ch, slab_rows, dstm_hbm, out_hbm, idx_d, ones_v, zbuf, acc):
    cid = lax.axis_index("c")
    sid = lax.axis_index("s")
    wid = cid * NS + sid
    slab = sid * slab_rows
    n_acc = slab_rows * NS

    _fill_const(zbuf, ZR, 0.0)
    _fill_const(ones_v, K, 1.0)

    @pl.loop(0, slab_rows // ZR)
    def _(i):
        pltpu.sync_copy(zbuf, acc.at[pl.ds(slab + i * ZR, ZR)])

    pltpu.sync_copy(dstm_hbm.at[pl.ds(wid * ch, ch)], idx_d)
    plsc.subcore_barrier()

    @pl.loop(0, ch)
    def _(c):
        pltpu.sync_copy(ones_v, acc.at[idx_d.at[c]], add=True)

    plsc.subcore_barrier()
    pltpu.sync_copy(acc.at[pl.ds(slab, slab_rows)],
                    out_hbm.at[pl.ds(cid * n_acc + slab, slab_rows)])


def _sc_pass_body(ch0, ch1, slab_rows, z_hbm, srcm_hbm, dstm_hbm, out_hbm,
                  is0, is1, id0, id1, rows0, rows1, zbuf, acc, sem0, sem1):
    cid = lax.axis_index("c")
    sid = lax.axis_index("s")
    slab = sid * slab_rows
    n_acc = slab_rows * NS
    ch = lax.select(cid == 0, ch0, ch1)
    base = lax.select(cid == 0, sid * ch0, NS * ch0 + sid * ch1)

    _fill_const(zbuf, ZR, 0.0)

    @pl.loop(0, slab_rows // ZR)
    def _(i):
        pltpu.sync_copy(zbuf, acc.at[pl.ds(slab + i * ZR, ZR)])

    pltpu.sync_copy(srcm_hbm.at[base], is0)
    pltpu.sync_copy(dstm_hbm.at[base], id0)
    pltpu.sync_copy(srcm_hbm.at[base + 1], is1)
    pltpu.sync_copy(dstm_hbm.at[base + 1], id1)
    pltpu.async_copy(z_hbm.at[is0], rows0, sem0)
    pltpu.async_copy(z_hbm.at[is1], rows1, sem1)

    plsc.subcore_barrier()

    @pl.loop(0, ch, step=2)
    def _(c):
        pltpu.make_async_copy(z_hbm.at[is0], rows0, sem0).wait()
        pltpu.sync_copy(rows0, acc.at[id0], add=True)

        @pl.when(c + 2 < ch)
        def _():
            pltpu.sync_copy(srcm_hbm.at[base + c + 2], is0)
            pltpu.sync_copy(dstm_hbm.at[base + c + 2], id0)
            pltpu.async_copy(z_hbm.at[is0], rows0, sem0)

        pltpu.make_async_copy(z_hbm.at[is1], rows1, sem1).wait()
        pltpu.sync_copy(rows1, acc.at[id1], add=True)

        @pl.when(c + 3 < ch)
        def _():
            pltpu.sync_copy(srcm_hbm.at[base + c + 3], is1)
            pltpu.sync_copy(dstm_hbm.at[base + c + 3], id1)
            pltpu.async_copy(z_hbm.at[is1], rows1, sem1)

    plsc.subcore_barrier()
    pltpu.sync_copy(acc.at[pl.ds(slab, slab_rows)],
                    out_hbm.at[pl.ds(cid * n_acc + slab, slab_rows)])


SLOW_CORE_FRAC = 0.5


@functools.cache
def _make_sc_kernels(n, e, d):
    chunks_total = -(-(-(-e // K)) // (NW * 8)) * (NW * 8)
    ch = chunks_total // NW
    ch_sum = chunks_total // NS
    ch1 = max(2, int(ch_sum * SLOW_CORE_FRAC) // 2 * 2)
    ch0 = ch_sum - ch1
    e_pad = chunks_total * K
    n_acc = -(-(n + 1) // (NS * ZR)) * (NS * ZR)
    slab_rows = n_acc // NS
    mesh = plsc.VectorSubcoreMesh(core_axis_name="c", subcore_axis_name="s")

    deg_kernel = pl.kernel(
        functools.partial(_sc_deg_body, ch, slab_rows),
        out_type=jax.ShapeDtypeStruct((NC * n_acc, 128), jnp.float32),
        mesh=mesh,
        scratch_types=[
            pltpu.VMEM((ch, K), jnp.int32),
            pltpu.VMEM((K, 128), jnp.float32),
            pltpu.VMEM((ZR, 128), jnp.float32),
            pltpu.VMEM_SHARED((n_acc, 128), jnp.float32),
        ],
    )

    pass_kernel = pl.kernel(
        functools.partial(_sc_pass_body, ch0, ch1, slab_rows),
        out_type=jax.ShapeDtypeStruct((NC * n_acc, d), jnp.float32),
        mesh=mesh,
        scratch_types=[
            pltpu.VMEM((K,), jnp.int32),
            pltpu.VMEM((K,), jnp.int32),
            pltpu.VMEM((K,), jnp.int32),
            pltpu.VMEM((K,), jnp.int32),
            pltpu.VMEM((K, d), jnp.float32),
            pltpu.VMEM((K, d), jnp.float32),
            pltpu.VMEM((ZR, 128), jnp.float32),
            pltpu.VMEM_SHARED((n_acc, d), jnp.float32),
            pltpu.SemaphoreType.DMA,
            pltpu.SemaphoreType.DMA,
        ],
    )
    return deg_kernel, pass_kernel, e_pad, n_acc


BLK = 2048


def _tc_mm1_body(x_ref, w_ref, o_ref):
    o_ref[...] = jnp.dot(x_ref[...], w_ref[...], preferred_element_type=jnp.float32)


def _tc_scale1_body(pa_ref, pb_ref, h1_ref, dinv_ref, z1_ref):
    dinv = lax.rsqrt(pa_ref[...] + pb_ref[...] + 1.0)
    dinv_ref[...] = dinv
    z1_ref[...] = dinv * h1_ref[...]


def _tc_mid_body(sa_ref, sb_ref, dinv_ref, h1_ref, b1_ref, hid_ref, z2_ref):
    dinv = dinv_ref[...]
    agg = dinv * (sa_ref[...] + sb_ref[...]) + dinv * dinv * h1_ref[...]
    hid = jnp.maximum(agg + b1_ref[...], 0.0)
    hid_ref[...] = hid
    z2_ref[...] = dinv * hid


def _tc_out_body(sa_ref, sb_ref, dinv_ref, hid_ref, wmu_ref, wlv_ref,
                 bmu_ref, blv_ref, mu_ref, lv_ref):
    dinv = dinv_ref[...]
    agg = dinv * (sa_ref[...] + sb_ref[...]) + dinv * dinv * hid_ref[...]
    mu_ref[...] = jnp.dot(agg, wmu_ref[...], preferred_element_type=jnp.float32) + bmu_ref[...]
    lv_ref[...] = jnp.dot(agg, wlv_ref[...], preferred_element_type=jnp.float32) + blv_ref[...]


def _row_spec(off_blocks):
    return pl.BlockSpec((BLK, 128), lambda i, _o=off_blocks: (i + _o, 0))


def _full_spec(shape):
    return pl.BlockSpec(shape, lambda i: (0, 0))


def kernel(x, adj, W1, b1, W_mu, b_mu, W_lv, b_lv):
    n, d = x.shape
    e = adj.shape[1]
    deg_kernel, pass_kernel, e_pad, n_acc = _make_sc_kernels(n, e, d)
    grid = (n_acc // BLK,)
    nb = n_acc // BLK
    h2 = W_mu.shape[1]

    src, dst = adj[0], adj[1]
    pad = e_pad - e
    pad_idx = jnp.arange(pad, dtype=jnp.int32) % 128
    srcm = jnp.concatenate([src, pad_idx]).reshape(e_pad // K, K)
    dstm = jnp.concatenate([dst, n + pad_idx]).reshape(e_pad // K, K)
    xp = jnp.concatenate([x, jnp.zeros((n_acc - n, d), jnp.float32)])

    degp = deg_kernel(dstm)
    h1 = pl.pallas_call(
        _tc_mm1_body, grid=grid,
        in_specs=[_row_spec(0), _full_spec((d, d))],
        out_specs=_row_spec(0),
        out_shape=jax.ShapeDtypeStruct((n_acc, d), jnp.float32),
    )(xp, W1)

    dinv, z1 = pl.pallas_call(
        _tc_scale1_body, grid=grid,
        in_specs=[_row_spec(0), _row_spec(nb), _row_spec(0)],
        out_specs=[_row_spec(0), _row_spec(0)],
        out_shape=[jax.ShapeDtypeStruct((n_acc, d), jnp.float32),
                   jax.ShapeDtypeStruct((n_acc, d), jnp.float32)],
    )(degp, degp, h1)

    s1 = pass_kernel(z1, srcm, dstm)

    hidden, z2 = pl.pallas_call(
        _tc_mid_body, grid=grid,
        in_specs=[_row_spec(0), _row_spec(nb), _row_spec(0), _row_spec(0),
                  pl.BlockSpec((1, d), lambda i: (0, 0))],
        out_specs=[_row_spec(0), _row_spec(0)],
        out_shape=[jax.ShapeDtypeStruct((n_acc, d), jnp.float32),
                   jax.ShapeDtypeStruct((n_acc, d), jnp.float32)],
    )(s1, s1, dinv, h1, b1.reshape(1, d))

    s2 = pass_kernel(z2, srcm, dstm)

    mu_f, lv_f = pl.pallas_call(
        _tc_out_body, grid=grid,
        in_specs=[_row_spec(0), _row_spec(nb), _row_spec(0), _row_spec(0),
                  _full_spec((d, h2)), _full_spec((d, h2)),
                  pl.BlockSpec((1, h2), lambda i: (0, 0)),
                  pl.BlockSpec((1, h2), lambda i: (0, 0))],
        out_specs=[pl.BlockSpec((BLK, h2), lambda i: (i, 0)),
                   pl.BlockSpec((BLK, h2), lambda i: (i, 0))],
        out_shape=[jax.ShapeDtypeStruct((n_acc, h2), jnp.float32),
                   jax.ShapeDtypeStruct((n_acc, h2), jnp.float32)],
    )(s2, s2, dinv, hidden, W_mu, W_lv, b_mu.reshape(1, h2), b_lv.reshape(1, h2))

    return (mu_f[:n], lv_f[:n])


# speedup vs baseline: 3.0675x; 1.0400x over previous
import functools
import jax
import jax.numpy as jnp
from jax import lax
from jax.experimental import pallas as pl
from jax.experimental.pallas import tpu as pltpu
from jax.experimental.pallas import tpu_sc as plsc

NC = 2
NS = 16
NW = NC * NS
K = 128
ZR = 32


def _fill_const(buf, rows, val):
    @pl.loop(0, rows)
    def _(r):
        @pl.loop(0, 8)
        def _(cc):
            buf[pl.ds(r, 1), pl.ds(cc * 16, 16)] = jnp.full((1, 16), val, jnp.float32)


def _zero_slab(zbuf, acc, slab, slab_rows, zsem):
    _fill_const(zbuf, ZR, 0.0)

    @pl.loop(0, slab_rows // ZR)
    def _(i):
        pltpu.async_copy(zbuf, acc.at[pl.ds(slab + i * ZR, ZR)], zsem)

    @pl.loop(0, slab_rows // ZR)
    def _(i):
        pltpu.make_async_copy(zbuf, acc.at[pl.ds(slab, ZR)], zsem).wait()


def _sc_deg_body(ch, slab_rows, dstm_hbm, out_hbm, id0, id1, ones_v, zbuf, acc,
                 isem0, isem1, ssem0, ssem1, zsem):
    cid = lax.axis_index("c")
    sid = lax.axis_index("s")
    wid = cid * NS + sid
    slab = sid * slab_rows
    n_acc = slab_rows * NS
    base = wid * ch

    _zero_slab(zbuf, acc, slab, slab_rows, zsem)
    _fill_const(ones_v, K, 1.0)

    pltpu.async_copy(dstm_hbm.at[base], id0, isem0)
    pltpu.async_copy(dstm_hbm.at[base + 1], id1, isem1)
    plsc.subcore_barrier()

    pltpu.make_async_copy(dstm_hbm.at[base], id0, isem0).wait()
    pltpu.async_copy(ones_v, acc.at[id0], ssem0, add=True)
    pltpu.make_async_copy(dstm_hbm.at[base], id1, isem1).wait()
    pltpu.async_copy(ones_v, acc.at[id1], ssem1, add=True)

    @pl.loop(0, ch, step=2)
    def _(c):
        @pl.when(c + 2 < ch)
        def _():
            pltpu.make_async_copy(ones_v, acc.at[id0], ssem0).wait()
            pltpu.async_copy(dstm_hbm.at[base + c + 2], id0, isem0)

        @pl.when(c + 3 < ch)
        def _():
            pltpu.make_async_copy(ones_v, acc.at[id1], ssem1).wait()
            pltpu.async_copy(dstm_hbm.at[base + c + 3], id1, isem1)

        @pl.when(c + 2 < ch)
        def _():
            pltpu.make_async_copy(dstm_hbm.at[base], id0, isem0).wait()
            pltpu.async_copy(ones_v, acc.at[id0], ssem0, add=True)

        @pl.when(c + 3 < ch)
        def _():
            pltpu.make_async_copy(dstm_hbm.at[base], id1, isem1).wait()
            pltpu.async_copy(ones_v, acc.at[id1], ssem1, add=True)

    pltpu.make_async_copy(ones_v, acc.at[id0], ssem0).wait()
    pltpu.make_async_copy(ones_v, acc.at[id1], ssem1).wait()

    plsc.subcore_barrier()
    pltpu.sync_copy(acc.at[pl.ds(slab, slab_rows)],
                    out_hbm.at[pl.ds(cid * n_acc + slab, slab_rows)])


def _sc_pass_body(ch0, ch1, slab_rows, z_hbm, srcm_hbm, dstm_hbm, out_hbm,
                  is0, is1, id0, id1, rows0, rows1, zbuf, acc,
                  isem0, isem1, gsem0, gsem1, zsem):
    cid = lax.axis_index("c")
    sid = lax.axis_index("s")
    slab = sid * slab_rows
    n_acc = slab_rows * NS
    ch = lax.select(cid == 0, ch0, ch1)
    base = lax.select(cid == 0, sid * ch0, NS * ch0 + sid * ch1)

    _zero_slab(zbuf, acc, slab, slab_rows, zsem)

    pltpu.async_copy(srcm_hbm.at[base], is0, isem0)
    pltpu.async_copy(dstm_hbm.at[base], id0, isem0)
    pltpu.async_copy(srcm_hbm.at[base + 1], is1, isem1)
    pltpu.async_copy(dstm_hbm.at[base + 1], id1, isem1)
    pltpu.make_async_copy(srcm_hbm.at[base], is0, isem0).wait()
    pltpu.make_async_copy(dstm_hbm.at[base], id0, isem0).wait()
    pltpu.async_copy(z_hbm.at[is0], rows0, gsem0)
    pltpu.make_async_copy(srcm_hbm.at[base], is1, isem1).wait()
    pltpu.make_async_copy(dstm_hbm.at[base], id1, isem1).wait()
    pltpu.async_copy(z_hbm.at[is1], rows1, gsem1)

    plsc.subcore_barrier()

    @pl.loop(0, ch, step=2)
    def _(c):
        pltpu.make_async_copy(z_hbm.at[is0], rows0, gsem0).wait()
        pltpu.sync_copy(rows0, acc.at[id0], add=True)

        @pl.when(c + 2 < ch)
        def _():
            pltpu.async_copy(srcm_hbm.at[base + c + 2], is0, isem0)
            pltpu.async_copy(dstm_hbm.at[base + c + 2], id0, isem0)

        pltpu.make_async_copy(z_hbm.at[is1], rows1, gsem1).wait()
        pltpu.sync_copy(rows1, acc.at[id1], add=True)

        @pl.when(c + 3 < ch)
        def _():
            pltpu.async_copy(srcm_hbm.at[base + c + 3], is1, isem1)
            pltpu.async_copy(dstm_hbm.at[base + c + 3], id1, isem1)

        @pl.when(c + 2 < ch)
        def _():
            pltpu.make_async_copy(srcm_hbm.at[base], is0, isem0).wait()
            pltpu.make_async_copy(dstm_hbm.at[base], id0, isem0).wait()
            pltpu.async_copy(z_hbm.at[is0], rows0, gsem0)

        @pl.when(c + 3 < ch)
        def _():
            pltpu.make_async_copy(srcm_hbm.at[base], is1, isem1).wait()
            pltpu.make_async_copy(dstm_hbm.at[base], id1, isem1).wait()
            pltpu.async_copy(z_hbm.at[is1], rows1, gsem1)

    plsc.subcore_barrier()
    pltpu.sync_copy(acc.at[pl.ds(slab, slab_rows)],
                    out_hbm.at[pl.ds(cid * n_acc + slab, slab_rows)])


SLOW_CORE_FRAC = 0.5


@functools.cache
def _make_sc_kernels(n, e, d):
    chunks_total = -(-(-(-e // K)) // (NW * 8)) * (NW * 8)
    ch = chunks_total // NW
    ch_sum = chunks_total // NS
    ch1 = max(2, int(ch_sum * SLOW_CORE_FRAC) // 2 * 2)
    ch0 = ch_sum - ch1
    e_pad = chunks_total * K
    n_acc = -(-(n + 1) // (NS * ZR)) * (NS * ZR)
    slab_rows = n_acc // NS
    mesh = plsc.VectorSubcoreMesh(core_axis_name="c", subcore_axis_name="s")

    deg_kernel = pl.kernel(
        functools.partial(_sc_deg_body, ch, slab_rows),
        out_type=jax.ShapeDtypeStruct((NC * n_acc, 128), jnp.float32),
        mesh=mesh,
        scratch_types=[
            pltpu.VMEM((K,), jnp.int32),
            pltpu.VMEM((K,), jnp.int32),
            pltpu.VMEM((K, 128), jnp.float32),
            pltpu.VMEM((ZR, 128), jnp.float32),
            pltpu.VMEM_SHARED((n_acc, 128), jnp.float32),
            pltpu.SemaphoreType.DMA,
            pltpu.SemaphoreType.DMA,
            pltpu.SemaphoreType.DMA,
            pltpu.SemaphoreType.DMA,
            pltpu.SemaphoreType.DMA,
        ],
    )

    pass_kernel = pl.kernel(
        functools.partial(_sc_pass_body, ch0, ch1, slab_rows),
        out_type=jax.ShapeDtypeStruct((NC * n_acc, d), jnp.float32),
        mesh=mesh,
        scratch_types=[
            pltpu.VMEM((K,), jnp.int32),
            pltpu.VMEM((K,), jnp.int32),
            pltpu.VMEM((K,), jnp.int32),
            pltpu.VMEM((K,), jnp.int32),
            pltpu.VMEM((K, d), jnp.float32),
            pltpu.VMEM((K, d), jnp.float32),
            pltpu.VMEM((ZR, 128), jnp.float32),
            pltpu.VMEM_SHARED((n_acc, d), jnp.float32),
            pltpu.SemaphoreType.DMA,
            pltpu.SemaphoreType.DMA,
            pltpu.SemaphoreType.DMA,
            pltpu.SemaphoreType.DMA,
            pltpu.SemaphoreType.DMA,
        ],
    )
    return deg_kernel, pass_kernel, e_pad, n_acc


BLK = 2048


def _tc_mm1_body(x_ref, w_ref, o_ref):
    o_ref[...] = jnp.dot(x_ref[...], w_ref[...], preferred_element_type=jnp.float32)


def _tc_scale1_body(pa_ref, pb_ref, h1_ref, dinv_ref, z1_ref):
    dinv = lax.rsqrt(pa_ref[...] + pb_ref[...] + 1.0)
    dinv_ref[...] = dinv
    z1_ref[...] = dinv * h1_ref[...]


def _tc_mid_body(sa_ref, sb_ref, dinv_ref, h1_ref, b1_ref, hid_ref, z2_ref):
    dinv = dinv_ref[...]
    agg = dinv * (sa_ref[...] + sb_ref[...]) + dinv * dinv * h1_ref[...]
    hid = jnp.maximum(agg + b1_ref[...], 0.0)
    hid_ref[...] = hid
    z2_ref[...] = dinv * hid


def _tc_out_body(sa_ref, sb_ref, dinv_ref, hid_ref, wmu_ref, wlv_ref,
                 bmu_ref, blv_ref, mu_ref, lv_ref):
    dinv = dinv_ref[...]
    agg = dinv * (sa_ref[...] + sb_ref[...]) + dinv * dinv * hid_ref[...]
    mu_ref[...] = jnp.dot(agg, wmu_ref[...], preferred_element_type=jnp.float32) + bmu_ref[...]
    lv_ref[...] = jnp.dot(agg, wlv_ref[...], preferred_element_type=jnp.float32) + blv_ref[...]


def _row_spec(off_blocks):
    return pl.BlockSpec((BLK, 128), lambda i, _o=off_blocks: (i + _o, 0))


def _full_spec(shape):
    return pl.BlockSpec(shape, lambda i: (0, 0))


def kernel(x, adj, W1, b1, W_mu, b_mu, W_lv, b_lv):
    n, d = x.shape
    e = adj.shape[1]
    deg_kernel, pass_kernel, e_pad, n_acc = _make_sc_kernels(n, e, d)
    grid = (n_acc // BLK,)
    nb = n_acc // BLK
    h2 = W_mu.shape[1]

    src, dst = adj[0], adj[1]
    pad = e_pad - e
    pad_idx = jnp.arange(pad, dtype=jnp.int32) % 128
    srcm = jnp.concatenate([src, pad_idx]).reshape(e_pad // K, K)
    dstm = jnp.concatenate([dst, n + pad_idx]).reshape(e_pad // K, K)
    xp = jnp.concatenate([x, jnp.zeros((n_acc - n, d), jnp.float32)])

    degp = deg_kernel(dstm)
    h1 = pl.pallas_call(
        _tc_mm1_body, grid=grid,
        in_specs=[_row_spec(0), _full_spec((d, d))],
        out_specs=_row_spec(0),
        out_shape=jax.ShapeDtypeStruct((n_acc, d), jnp.float32),
    )(xp, W1)

    dinv, z1 = pl.pallas_call(
        _tc_scale1_body, grid=grid,
        in_specs=[_row_spec(0), _row_spec(nb), _row_spec(0)],
        out_specs=[_row_spec(0), _row_spec(0)],
        out_shape=[jax.ShapeDtypeStruct((n_acc, d), jnp.float32),
                   jax.ShapeDtypeStruct((n_acc, d), jnp.float32)],
    )(degp, degp, h1)

    s1 = pass_kernel(z1, srcm, dstm)

    hidden, z2 = pl.pallas_call(
        _tc_mid_body, grid=grid,
        in_specs=[_row_spec(0), _row_spec(nb), _row_spec(0), _row_spec(0),
                  pl.BlockSpec((1, d), lambda i: (0, 0))],
        out_specs=[_row_spec(0), _row_spec(0)],
        out_shape=[jax.ShapeDtypeStruct((n_acc, d), jnp.float32),
                   jax.ShapeDtypeStruct((n_acc, d), jnp.float32)],
    )(s1, s1, dinv, h1, b1.reshape(1, d))

    s2 = pass_kernel(z2, srcm, dstm)

    mu_f, lv_f = pl.pallas_call(
        _tc_out_body, grid=grid,
        in_specs=[_row_spec(0), _row_spec(nb), _row_spec(0), _row_spec(0),
                  _full_spec((d, h2)), _full_spec((d, h2)),
                  pl.BlockSpec((1, h2), lambda i: (0, 0)),
                  pl.BlockSpec((1, h2), lambda i: (0, 0))],
        out_specs=[pl.BlockSpec((BLK, h2), lambda i: (i, 0)),
                   pl.BlockSpec((BLK, h2), lambda i: (i, 0))],
        out_shape=[jax.ShapeDtypeStruct((n_acc, h2), jnp.float32),
                   jax.ShapeDtypeStruct((n_acc, h2), jnp.float32)],
    )(s2, s2, dinv, hidden, W_mu, W_lv, b_mu.reshape(1, h2), b_lv.reshape(1, h2))

    return (mu_f[:n], lv_f[:n])


# speedup vs baseline: 3.7061x; 1.2082x over previous
import functools
import jax
import jax.numpy as jnp
from jax import lax
from jax.experimental import pallas as pl
from jax.experimental.pallas import tpu as pltpu
from jax.experimental.pallas import tpu_sc as plsc

NC = 2
NS = 16
NW = NC * NS
K = 128
ZR = 32


def _fill_const(buf, rows, val):
    @pl.loop(0, rows)
    def _(r):
        @pl.loop(0, 8)
        def _(cc):
            buf[pl.ds(r, 1), pl.ds(cc * 16, 16)] = jnp.full((1, 16), val, jnp.float32)


def _zero_slab(zbuf, acc, slab, slab_rows, zsem):
    _fill_const(zbuf, ZR, 0.0)

    @pl.loop(0, slab_rows // ZR)
    def _(i):
        pltpu.async_copy(zbuf, acc.at[pl.ds(slab + i * ZR, ZR)], zsem)

    @pl.loop(0, slab_rows // ZR)
    def _(i):
        pltpu.make_async_copy(zbuf, acc.at[pl.ds(slab, ZR)], zsem).wait()


def _sc_deg_body(ch, slab_rows, dstm_hbm, out_hbm, id0, id1, ones_v, zbuf, acc,
                 isem0, isem1, zsem):
    cid = lax.axis_index("c")
    sid = lax.axis_index("s")
    wid = cid * NS + sid
    slab = sid * slab_rows
    n_acc = slab_rows * NS
    base = wid * ch

    _zero_slab(zbuf, acc, slab, slab_rows, zsem)
    _fill_const(ones_v, K, 1.0)

    pltpu.async_copy(dstm_hbm.at[base], id0, isem0)
    pltpu.async_copy(dstm_hbm.at[base + 1], id1, isem1)
    plsc.subcore_barrier()

    @pl.loop(0, ch, step=2)
    def _(c):
        pltpu.make_async_copy(dstm_hbm.at[base], id0, isem0).wait()
        pltpu.sync_copy(ones_v, acc.at[id0], add=True)

        @pl.when(c + 2 < ch)
        def _():
            pltpu.async_copy(dstm_hbm.at[base + c + 2], id0, isem0)

        pltpu.make_async_copy(dstm_hbm.at[base], id1, isem1).wait()
        pltpu.sync_copy(ones_v, acc.at[id1], add=True)

        @pl.when(c + 3 < ch)
        def _():
            pltpu.async_copy(dstm_hbm.at[base + c + 3], id1, isem1)

    plsc.subcore_barrier()
    pltpu.sync_copy(acc.at[pl.ds(slab, slab_rows)],
                    out_hbm.at[pl.ds(cid * n_acc + slab, slab_rows)])


def _sc_pass_body(ch0, ch1, slab_rows, z_hbm, srcm_hbm, dstm_hbm, out_hbm,
                  is0, is1, id0, id1, rows0, rows1, zbuf, acc,
                  isem_s0, isem_s1, isem_d0, isem_d1, gsem0, gsem1, zsem):
    cid = lax.axis_index("c")
    sid = lax.axis_index("s")
    slab = sid * slab_rows
    n_acc = slab_rows * NS
    ch = lax.select(cid == 0, ch0, ch1)
    base = lax.select(cid == 0, sid * ch0, NS * ch0 + sid * ch1)

    _zero_slab(zbuf, acc, slab, slab_rows, zsem)

    pltpu.async_copy(srcm_hbm.at[base], is0, isem_s0)
    pltpu.async_copy(dstm_hbm.at[base], id0, isem_d0)
    pltpu.async_copy(srcm_hbm.at[base + 1], is1, isem_s1)
    pltpu.async_copy(dstm_hbm.at[base + 1], id1, isem_d1)
    pltpu.make_async_copy(srcm_hbm.at[base], is0, isem_s0).wait()
    pltpu.async_copy(z_hbm.at[is0], rows0, gsem0)
    pltpu.make_async_copy(srcm_hbm.at[base], is1, isem_s1).wait()
    pltpu.async_copy(z_hbm.at[is1], rows1, gsem1)

    plsc.subcore_barrier()

    @pl.loop(0, ch, step=2)
    def _(c):
        pltpu.make_async_copy(z_hbm.at[is0], rows0, gsem0).wait()

        @pl.when(c + 2 < ch)
        def _():
            pltpu.async_copy(srcm_hbm.at[base + c + 2], is0, isem_s0)

        pltpu.make_async_copy(dstm_hbm.at[base], id0, isem_d0).wait()
        pltpu.sync_copy(rows0, acc.at[id0], add=True)

        @pl.when(c + 2 < ch)
        def _():
            pltpu.async_copy(dstm_hbm.at[base + c + 2], id0, isem_d0)
            pltpu.make_async_copy(srcm_hbm.at[base], is0, isem_s0).wait()
            pltpu.async_copy(z_hbm.at[is0], rows0, gsem0)

        pltpu.make_async_copy(z_hbm.at[is1], rows1, gsem1).wait()

        @pl.when(c + 3 < ch)
        def _():
            pltpu.async_copy(srcm_hbm.at[base + c + 3], is1, isem_s1)

        pltpu.make_async_copy(dstm_hbm.at[base], id1, isem_d1).wait()
        pltpu.sync_copy(rows1, acc.at[id1], add=True)

        @pl.when(c + 3 < ch)
        def _():
            pltpu.async_copy(dstm_hbm.at[base + c + 3], id1, isem_d1)
            pltpu.make_async_copy(srcm_hbm.at[base], is1, isem_s1).wait()
            pltpu.async_copy(z_hbm.at[is1], rows1, gsem1)

    plsc.subcore_barrier()
    pltpu.sync_copy(acc.at[pl.ds(slab, slab_rows)],
                    out_hbm.at[pl.ds(cid * n_acc + slab, slab_rows)])


SLOW_CORE_FRAC = 0.5


@functools.cache
def _make_sc_kernels(n, e, d):
    chunks_total = -(-(-(-e // K)) // (NW * 8)) * (NW * 8)
    ch = chunks_total // NW
    ch_sum = chunks_total // NS
    ch1 = max(2, int(ch_sum * SLOW_CORE_FRAC) // 2 * 2)
    ch0 = ch_sum - ch1
    e_pad = chunks_total * K
    n_acc = -(-(n + 1) // (NS * ZR)) * (NS * ZR)
    slab_rows = n_acc // NS
    mesh = plsc.VectorSubcoreMesh(core_axis_name="c", subcore_axis_name="s")

    deg_kernel = pl.kernel(
        functools.partial(_sc_deg_body, ch, slab_rows),
        out_type=jax.ShapeDtypeStruct((NC * n_acc, 128), jnp.float32),
        mesh=mesh,
        scratch_types=[
            pltpu.VMEM((K,), jnp.int32),
            pltpu.VMEM((K,), jnp.int32),
            pltpu.VMEM((K, 128), jnp.float32),
            pltpu.VMEM((ZR, 128), jnp.float32),
            pltpu.VMEM_SHARED((n_acc, 128), jnp.float32),
            pltpu.SemaphoreType.DMA,
            pltpu.SemaphoreType.DMA,
            pltpu.SemaphoreType.DMA,
        ],
    )

    pass_kernel = pl.kernel(
        functools.partial(_sc_pass_body, ch0, ch1, slab_rows),
        out_type=jax.ShapeDtypeStruct((NC * n_acc, d), jnp.float32),
        mesh=mesh,
        scratch_types=[
            pltpu.VMEM((K,), jnp.int32),
            pltpu.VMEM((K,), jnp.int32),
            pltpu.VMEM((K,), jnp.int32),
            pltpu.VMEM((K,), jnp.int32),
            pltpu.VMEM((K, d), jnp.float32),
            pltpu.VMEM((K, d), jnp.float32),
            pltpu.VMEM((ZR, 128), jnp.float32),
            pltpu.VMEM_SHARED((n_acc, d), jnp.float32),
            pltpu.SemaphoreType.DMA,
            pltpu.SemaphoreType.DMA,
            pltpu.SemaphoreType.DMA,
            pltpu.SemaphoreType.DMA,
            pltpu.SemaphoreType.DMA,
            pltpu.SemaphoreType.DMA,
            pltpu.SemaphoreType.DMA,
        ],
    )
    return deg_kernel, pass_kernel, e_pad, n_acc


BLK = 2048


def _tc_mm1_body(x_ref, w_ref, o_ref):
    o_ref[...] = jnp.dot(x_ref[...], w_ref[...], preferred_element_type=jnp.float32)


def _tc_scale1_body(pa_ref, pb_ref, h1_ref, dinv_ref, z1_ref):
    dinv = lax.rsqrt(pa_ref[...] + pb_ref[...] + 1.0)
    dinv_ref[...] = dinv
    z1_ref[...] = dinv * h1_ref[...]


def _tc_mid_body(sa_ref, sb_ref, dinv_ref, h1_ref, b1_ref, hid_ref, z2_ref):
    dinv = dinv_ref[...]
    agg = dinv * (sa_ref[...] + sb_ref[...]) + dinv * dinv * h1_ref[...]
    hid = jnp.maximum(agg + b1_ref[...], 0.0)
    hid_ref[...] = hid
    z2_ref[...] = dinv * hid


def _tc_out_body(sa_ref, sb_ref, dinv_ref, hid_ref, wmu_ref, wlv_ref,
                 bmu_ref, blv_ref, mu_ref, lv_ref):
    dinv = dinv_ref[...]
    agg = dinv * (sa_ref[...] + sb_ref[...]) + dinv * dinv * hid_ref[...]
    mu_ref[...] = jnp.dot(agg, wmu_ref[...], preferred_element_type=jnp.float32) + bmu_ref[...]
    lv_ref[...] = jnp.dot(agg, wlv_ref[...], preferred_element_type=jnp.float32) + blv_ref[...]


def _row_spec(off_blocks):
    return pl.BlockSpec((BLK, 128), lambda i, _o=off_blocks: (i + _o, 0))


def _full_spec(shape):
    return pl.BlockSpec(shape, lambda i: (0, 0))


def kernel(x, adj, W1, b1, W_mu, b_mu, W_lv, b_lv):
    n, d = x.shape
    e = adj.shape[1]
    deg_kernel, pass_kernel, e_pad, n_acc = _make_sc_kernels(n, e, d)
    grid = (n_acc // BLK,)
    nb = n_acc // BLK
    h2 = W_mu.shape[1]

    src, dst = adj[0], adj[1]
    pad = e_pad - e
    pad_idx = jnp.arange(pad, dtype=jnp.int32) % 128
    srcm = jnp.concatenate([src, pad_idx]).reshape(e_pad // K, K)
    dstm = jnp.concatenate([dst, n + pad_idx]).reshape(e_pad // K, K)
    xp = jnp.concatenate([x, jnp.zeros((n_acc - n, d), jnp.float32)])

    degp = deg_kernel(dstm)
    h1 = pl.pallas_call(
        _tc_mm1_body, grid=grid,
        in_specs=[_row_spec(0), _full_spec((d, d))],
        out_specs=_row_spec(0),
        out_shape=jax.ShapeDtypeStruct((n_acc, d), jnp.float32),
    )(xp, W1)

    dinv, z1 = pl.pallas_call(
        _tc_scale1_body, grid=grid,
        in_specs=[_row_spec(0), _row_spec(nb), _row_spec(0)],
        out_specs=[_row_spec(0), _row_spec(0)],
        out_shape=[jax.ShapeDtypeStruct((n_acc, d), jnp.float32),
                   jax.ShapeDtypeStruct((n_acc, d), jnp.float32)],
    )(degp, degp, h1)

    s1 = pass_kernel(z1, srcm, dstm)

    hidden, z2 = pl.pallas_call(
        _tc_mid_body, grid=grid,
        in_specs=[_row_spec(0), _row_spec(nb), _row_spec(0), _row_spec(0),
                  pl.BlockSpec((1, d), lambda i: (0, 0))],
        out_specs=[_row_spec(0), _row_spec(0)],
        out_shape=[jax.ShapeDtypeStruct((n_acc, d), jnp.float32),
                   jax.ShapeDtypeStruct((n_acc, d), jnp.float32)],
    )(s1, s1, dinv, h1, b1.reshape(1, d))

    s2 = pass_kernel(z2, srcm, dstm)

    mu_f, lv_f = pl.pallas_call(
        _tc_out_body, grid=grid,
        in_specs=[_row_spec(0), _row_spec(nb), _row_spec(0), _row_spec(0),
                  _full_spec((d, h2)), _full_spec((d, h2)),
                  pl.BlockSpec((1, h2), lambda i: (0, 0)),
                  pl.BlockSpec((1, h2), lambda i: (0, 0))],
        out_specs=[pl.BlockSpec((BLK, h2), lambda i: (i, 0)),
                   pl.BlockSpec((BLK, h2), lambda i: (i, 0))],
        out_shape=[jax.ShapeDtypeStruct((n_acc, h2), jnp.float32),
                   jax.ShapeDtypeStruct((n_acc, h2), jnp.float32)],
    )(s2, s2, dinv, hidden, W_mu, W_lv, b_mu.reshape(1, h2), b_lv.reshape(1, h2))

    return (mu_f[:n], lv_f[:n])
